# baseline (speedup 1.0000x reference)
import functools

import jax
import jax.numpy as jnp
from jax import lax
from jax.experimental import pallas as pl
from jax.experimental.pallas import tpu as pltpu
from jax.experimental.pallas import tpu_sc as plsc

SEQ = 32768
BLK = 128
NB = SEQ // BLK
L = 16
NC = 2
NS = 16
ROWS_PER_WORKER = NB // NS
HALF = NB // 2


def _body(doc_hbm, kv_num_hbm, kv_idx_hbm, q_num_hbm, q_idx_hbm,
          s_lo_v, s_hi_v, e_lo_v, e_hi_v, ds_v, de_v, nbuf_v, mbuf_v,
          kvrows_v, qrows_v, sem):
    iota = lax.iota(jnp.int32, L)
    wid = lax.axis_index("s")

    for j in range(HALF // L):
        lo = (iota + (L * j)) * BLK
        hi = (iota + (L * j + HALF)) * BLK
        s_lo_v[pl.ds(L * j, L)] = lo
        s_hi_v[pl.ds(L * j, L)] = hi
        e_lo_v[pl.ds(L * j, L)] = lo + (BLK - 1)
        e_hi_v[pl.ds(L * j, L)] = hi + (BLK - 1)

    c1 = pltpu.async_copy(doc_hbm.at[s_lo_v], ds_v.at[pl.ds(0, HALF)], sem)
    c2 = pltpu.async_copy(doc_hbm.at[s_hi_v], ds_v.at[pl.ds(HALF, HALF)], sem)
    c3 = pltpu.async_copy(doc_hbm.at[e_lo_v], de_v.at[pl.ds(0, HALF)], sem)
    c4 = pltpu.async_copy(doc_hbm.at[e_hi_v], de_v.at[pl.ds(HALF, HALF)], sem)
    c1.wait()
    c2.wait()
    c3.wait()
    c4.wait()

    q0 = wid * ROWS_PER_WORKER
    qv = q0 + iota
    de_lanes = plsc.load_gather(de_v, [qv])
    ds_lanes = plsc.load_gather(ds_v, [qv])
    lo1 = jnp.zeros((L,), jnp.int32)
    hi1 = jnp.full((L,), NB, jnp.int32)
    lo2 = jnp.zeros((L,), jnp.int32)
    hi2 = jnp.full((L,), NB, jnp.int32)
    for _ in range(8):
        mid1 = jnp.right_shift(lo1 + hi1, 1)
        v1 = plsc.load_gather(ds_v, [mid1])
        k1 = v1 <= de_lanes
        lo1 = jnp.where(k1, mid1 + 1, lo1)
        hi1 = jnp.where(k1, hi1, mid1)
        mid2 = jnp.right_shift(lo2 + hi2, 1)
        v2 = plsc.load_gather(de_v, [mid2])
        k2 = v2 < ds_lanes
        lo2 = jnp.where(k2, mid2 + 1, lo2)
        hi2 = jnp.where(k2, hi2, mid2)
    n_vec = lo1 - qv
    m_vec = qv - lo2 + 1
    nbuf_v[pl.ds(L, L)] = n_vec
    mbuf_v[pl.ds(L, L)] = m_vec
    cn = pltpu.async_copy(nbuf_v.at[pl.ds(L, L)],
                          kv_num_hbm.at[0, 0, pl.ds(q0, L)], sem)
    cm = pltpu.async_copy(mbuf_v.at[pl.ds(L, L)],
                          q_num_hbm.at[0, 0, pl.ds(q0, L)], sem)

    @pl.loop(0, ROWS_PER_WORKER)
    def _kv_row(l):
        qb = q0 + l
        n = plsc.load_gather(nbuf_v, [jnp.full((L,), L, jnp.int32) + l])
        t = n + qb

        @pl.loop(0, NB // L, unroll=8)
        def _chunk(j):
            p = iota + L * j
            val = jnp.where(p < n, qb + p,
                            jnp.where(p < t, p - n, p))
            kvrows_v[l, pl.ds(L * j, L)] = val

    ckv = pltpu.async_copy(kvrows_v,
                           kv_idx_hbm.at[0, 0, pl.ds(q0, ROWS_PER_WORKER), :],
                           sem)

    @pl.loop(0, ROWS_PER_WORKER)
    def _q_row(l):
        kb = q0 + l
        m = plsc.load_gather(mbuf_v, [jnp.full((L,), L, jnp.int32) + l])
        u = kb - m + 1

        @pl.loop(0, NB // L, unroll=8)
        def _chunk(j):
            p = iota + L * j
            val = jnp.where(p < m, u + p,
                            jnp.where(p < kb + 1, p - m, p))
            qrows_v[l, pl.ds(L * j, L)] = val

    cq = pltpu.async_copy(qrows_v,
                          q_idx_hbm.at[0, 0, pl.ds(q0, ROWS_PER_WORKER), :],
                          sem)
    cn.wait()
    cm.wait()
    ckv.wait()
    cq.wait()


_block_mask_sc = functools.partial(
    pl.kernel,
    mesh=plsc.VectorSubcoreMesh(core_axis_name="c", subcore_axis_name="s", num_cores=1),
    compiler_params=pltpu.CompilerParams(needs_layout_passes=False),
    out_type=[
        jax.ShapeDtypeStruct((1, 1, NB), jnp.int32),
        jax.ShapeDtypeStruct((1, 1, NB, NB), jnp.int32),
        jax.ShapeDtypeStruct((1, 1, NB), jnp.int32),
        jax.ShapeDtypeStruct((1, 1, NB, NB), jnp.int32),
    ],
    scratch_types=[
        pltpu.VMEM((HALF,), jnp.int32),
        pltpu.VMEM((HALF,), jnp.int32),
        pltpu.VMEM((HALF,), jnp.int32),
        pltpu.VMEM((HALF,), jnp.int32),
        pltpu.VMEM((NB,), jnp.int32),
        pltpu.VMEM((NB,), jnp.int32),
        pltpu.VMEM((2 * L,), jnp.int32),
        pltpu.VMEM((2 * L,), jnp.int32),
        pltpu.VMEM((ROWS_PER_WORKER, NB), jnp.int32),
        pltpu.VMEM((ROWS_PER_WORKER, NB), jnp.int32),
        pltpu.SemaphoreType.DMA,
    ],
)(_body)


def kernel(x, document_id):
    del x
    kv_num, kv_idx, q_num, q_idx = _block_mask_sc(document_id)
    blk = jnp.array([BLK], dtype=jnp.int32)
    return (kv_num, kv_idx, q_num, q_idx, blk, blk)


# speedup vs baseline: 1.0126x; 1.0126x over previous
import functools

import jax
import jax.numpy as jnp
from jax import lax
from jax.experimental import pallas as pl
from jax.experimental.pallas import tpu as pltpu
from jax.experimental.pallas import tpu_sc as plsc

SEQ = 32768
BLK = 128
NB = SEQ // BLK
L = 16
NC = 2
NS = 16
ROWS_PER_WORKER = NB // NS
HALF = NB // 2


def _body(doc_hbm, kv_num_hbm, kv_idx_hbm, q_num_hbm, q_idx_hbm,
          s_lo_v, s_hi_v, e_lo_v, e_hi_v, ds_v, de_v, nbuf_v, mbuf_v,
          kvrows_v, qrows_v, sem):
    iota = lax.iota(jnp.int32, L)
    wid = lax.axis_index("s")

    for j in range(HALF // L):
        lo = (iota + (L * j)) * BLK
        hi = (iota + (L * j + HALF)) * BLK
        s_lo_v[pl.ds(L * j, L)] = lo
        s_hi_v[pl.ds(L * j, L)] = hi
        e_lo_v[pl.ds(L * j, L)] = lo + (BLK - 1)
        e_hi_v[pl.ds(L * j, L)] = hi + (BLK - 1)

    c1 = pltpu.async_copy(doc_hbm.at[s_lo_v], ds_v.at[pl.ds(0, HALF)], sem)
    c2 = pltpu.async_copy(doc_hbm.at[s_hi_v], ds_v.at[pl.ds(HALF, HALF)], sem)
    c3 = pltpu.async_copy(doc_hbm.at[e_lo_v], de_v.at[pl.ds(0, HALF)], sem)
    c4 = pltpu.async_copy(doc_hbm.at[e_hi_v], de_v.at[pl.ds(HALF, HALF)], sem)
    c1.wait()
    c2.wait()
    c3.wait()
    c4.wait()

    q0 = wid * ROWS_PER_WORKER
    qv = q0 + iota
    de_lanes = plsc.load_gather(de_v, [qv])
    ds_lanes = plsc.load_gather(ds_v, [qv])
    lo1 = jnp.zeros((L,), jnp.int32)
    hi1 = jnp.full((L,), NB, jnp.int32)
    lo2 = jnp.zeros((L,), jnp.int32)
    hi2 = jnp.full((L,), NB, jnp.int32)
    for _ in range(8):
        mid1 = jnp.right_shift(lo1 + hi1, 1)
        v1 = plsc.load_gather(ds_v, [mid1])
        k1 = v1 <= de_lanes
        lo1 = jnp.where(k1, mid1 + 1, lo1)
        hi1 = jnp.where(k1, hi1, mid1)
        mid2 = jnp.right_shift(lo2 + hi2, 1)
        v2 = plsc.load_gather(de_v, [mid2])
        k2 = v2 < ds_lanes
        lo2 = jnp.where(k2, mid2 + 1, lo2)
        hi2 = jnp.where(k2, hi2, mid2)
    n_vec = lo1 - qv
    m_vec = qv - lo2 + 1
    nbuf_v[pl.ds(L, L)] = n_vec
    mbuf_v[pl.ds(L, L)] = m_vec
    cn = pltpu.async_copy(nbuf_v.at[pl.ds(L, L)],
                          kv_num_hbm.at[0, 0, pl.ds(q0, L)], sem)
    cm = pltpu.async_copy(mbuf_v.at[pl.ds(L, L)],
                          q_num_hbm.at[0, 0, pl.ds(q0, L)], sem)

    @pl.loop(0, ROWS_PER_WORKER)
    def _kv_row(l):
        qb = q0 + l
        n = plsc.load_gather(nbuf_v, [jnp.full((L,), L, jnp.int32) + l])
        t = n + qb

        @pl.loop(0, NB // L, unroll=4)
        def _chunk(j):
            p = iota + L * j
            val = jnp.where(p < n, qb + p,
                            jnp.where(p < t, p - n, p))
            kvrows_v[l, pl.ds(L * j, L)] = val

    ckv = pltpu.async_copy(kvrows_v,
                           kv_idx_hbm.at[0, 0, pl.ds(q0, ROWS_PER_WORKER), :],
                           sem)

    @pl.loop(0, ROWS_PER_WORKER)
    def _q_row(l):
        kb = q0 + l
        m = plsc.load_gather(mbuf_v, [jnp.full((L,), L, jnp.int32) + l])
        u = kb - m + 1

        @pl.loop(0, NB // L, unroll=4)
        def _chunk(j):
            p = iota + L * j
            val = jnp.where(p < m, u + p,
                            jnp.where(p < kb + 1, p - m, p))
            qrows_v[l, pl.ds(L * j, L)] = val

    cq = pltpu.async_copy(qrows_v,
                          q_idx_hbm.at[0, 0, pl.ds(q0, ROWS_PER_WORKER), :],
                          sem)
    cn.wait()
    cm.wait()
    ckv.wait()
    cq.wait()


_block_mask_sc = functools.partial(
    pl.kernel,
    mesh=plsc.VectorSubcoreMesh(core_axis_name="c", subcore_axis_name="s", num_cores=1),
    compiler_params=pltpu.CompilerParams(needs_layout_passes=False),
    out_type=[
        jax.ShapeDtypeStruct((1, 1, NB), jnp.int32),
        jax.ShapeDtypeStruct((1, 1, NB, NB), jnp.int32),
        jax.ShapeDtypeStruct((1, 1, NB), jnp.int32),
        jax.ShapeDtypeStruct((1, 1, NB, NB), jnp.int32),
    ],
    scratch_types=[
        pltpu.VMEM((HALF,), jnp.int32),
        pltpu.VMEM((HALF,), jnp.int32),
        pltpu.VMEM((HALF,), jnp.int32),
        pltpu.VMEM((HALF,), jnp.int32),
        pltpu.VMEM((NB,), jnp.int32),
        pltpu.VMEM((NB,), jnp.int32),
        pltpu.VMEM((2 * L,), jnp.int32),
        pltpu.VMEM((2 * L,), jnp.int32),
        pltpu.VMEM((ROWS_PER_WORKER, NB), jnp.int32),
        pltpu.VMEM((ROWS_PER_WORKER, NB), jnp.int32),
        pltpu.SemaphoreType.DMA,
    ],
)(_body)


def kernel(x, document_id):
    del x
    kv_num, kv_idx, q_num, q_idx = _block_mask_sc(document_id)
    blk = jnp.array([BLK], dtype=jnp.int32)
    return (kv_num, kv_idx, q_num, q_idx, blk, blk)
